# baseline (speedup 1.0000x reference)
import functools
import jax
import jax.numpy as jnp
from jax import lax
from jax.experimental import pallas as pl
from jax.experimental.pallas import tpu as pltpu
from jax.experimental.pallas import tpu_sc as plsc

N_NODES = 10000
H = 128
E = 320000
NC = 2
NS = 16
NW = NC * NS
EPW = E // NW
C = 160
NCHUNK = EPW // C
TAIL = EPW - NCHUNK * C
NPAIR = NCHUNK // 2
G = C // 16


def _tc_precompute(h, AT, BT, b1row):
    BLK = 1000

    def body(h_ref, at_ref, bt_ref, b1_ref, u_ref, v_ref):
        hb = h_ref[...]
        u_ref[...] = jnp.dot(hb, at_ref[...], preferred_element_type=jnp.float32)
        v_ref[...] = (jnp.dot(hb, bt_ref[...], preferred_element_type=jnp.float32)
                      + b1_ref[...])

    return pl.pallas_call(
        body,
        grid=(N_NODES // BLK,),
        in_specs=[
            pl.BlockSpec((BLK, H), lambda i: (i, 0)),
            pl.BlockSpec((H, H), lambda i: (0, 0)),
            pl.BlockSpec((H, H), lambda i: (0, 0)),
            pl.BlockSpec((1, H), lambda i: (0, 0)),
        ],
        out_specs=[
            pl.BlockSpec((BLK, H), lambda i: (i, 0)),
            pl.BlockSpec((BLK, H), lambda i: (i, 0)),
        ],
        out_shape=[
            jax.ShapeDtypeStruct((N_NODES, H), jnp.float32),
            jax.ShapeDtypeStruct((N_NODES, H), jnp.float32),
        ],
    )(h, AT, BT, b1row)


def _make_sc_kernel():
    mesh = plsc.VectorSubcoreMesh(core_axis_name="c", subcore_axis_name="s")

    @functools.partial(
        pl.kernel,
        out_type=jax.ShapeDtypeStruct((E,), jnp.float32),
        mesh=mesh,
        compiler_params=pltpu.CompilerParams(needs_layout_passes=False),
        scratch_types=[
            pltpu.VMEM((EPW,), jnp.int32),
            pltpu.VMEM((EPW,), jnp.int32),
            pltpu.VMEM((C, H), jnp.float32),
            pltpu.VMEM((C, H), jnp.float32),
            pltpu.VMEM((C, H), jnp.float32),
            pltpu.VMEM((C, H), jnp.float32),
            pltpu.VMEM((2 * C,), jnp.float32),
            pltpu.VMEM((G * 256,), jnp.float32),
            pltpu.VMEM((H,), jnp.float32),
            pltpu.VMEM((16,), jnp.float32),
            pltpu.SemaphoreType.DMA,
            pltpu.SemaphoreType.DMA,
            pltpu.SemaphoreType.DMA,
            pltpu.SemaphoreType.DMA,
        ],
    )
    def sc_kernel(u_hbm, v_hbm, src_hbm, dst_hbm, w2_hbm, b2_hbm, out_hbm,
                  src_all, dst_all, u0, v0, u1, v1, scores_v, red_flat,
                  w2_v, b2_v, su0, sv0, su1, sv1):
        wid = lax.axis_index("s") * NC + lax.axis_index("c")
        base = wid * EPW

        pltpu.sync_copy(src_hbm.at[pl.ds(base, EPW)], src_all)
        pltpu.sync_copy(dst_hbm.at[pl.ds(base, EPW)], dst_all)
        pltpu.sync_copy(w2_hbm, w2_v)
        pltpu.sync_copy(b2_hbm, b2_v)
        w2v = [w2_v[pl.ds(16 * j, 16)] for j in range(8)]
        b2vec = b2_v[...]
        lane = lax.iota(jnp.int32, 16)

        def prefetch(c, n, ubuf, vbuf, su, sv):
            sl = src_all.at[pl.ds(c * C, n)]
            dl = dst_all.at[pl.ds(c * C, n)]
            pltpu.async_copy(u_hbm.at[sl], ubuf, su)
            pltpu.async_copy(v_hbm.at[dl], vbuf, sv)

        def wait_set(ubuf, vbuf, su, sv):
            pltpu.make_async_copy(u_hbm.at[src_all.at[pl.ds(0, C)]], ubuf, su).wait()
            pltpu.make_async_copy(v_hbm.at[dst_all.at[pl.ds(0, C)]], vbuf, sv).wait()

        def compute(n, ubuf, vbuf, soff, unroll=4):
            @plsc.parallel_loop(0, n // 16, unroll=unroll)
            def group_body(g):
                gbase = g * 16
                roff = g * 256
                for e in range(16):
                    row = gbase + e
                    acc = None
                    for j in range(8):
                        u = ubuf[row, pl.ds(16 * j, 16)]
                        v = vbuf[row, pl.ds(16 * j, 16)]
                        t = jnp.maximum(u + v, 0.0) * w2v[j]
                        acc = t if acc is None else acc + t
                    red_flat[pl.ds(roff + e * 16, 16)] = acc
                idx = lane * 16 + roff
                s0 = plsc.load_gather(red_flat, [idx])
                s1 = None
                for j in range(1, 16):
                    idx = idx + 1
                    t = plsc.load_gather(red_flat, [idx])
                    if j % 2:
                        s1 = t if s1 is None else s1 + t
                    else:
                        s0 = s0 + t
                scores_v[pl.ds(soff + gbase, 16)] = s0 + s1 + b2vec

        def store_scores(c, n):
            pltpu.sync_copy(scores_v.at[pl.ds(0, n)],
                            out_hbm.at[pl.ds(base + c * C, n)])

        prefetch(0, C, u0, v0, su0, sv0)

        def pair_body(i, carry):
            c0 = i * 2
            prefetch(c0 + 1, C, u1, v1, su1, sv1)
            wait_set(u0, v0, su0, sv0)
            compute(C, u0, v0, 0)
            cn = lax.rem(c0 + 2, NCHUNK)
            prefetch(cn, C, u0, v0, su0, sv0)
            wait_set(u1, v1, su1, sv1)
            compute(C, u1, v1, C)
            store_scores(c0, 2 * C)
            return carry

        lax.fori_loop(0, NPAIR, pair_body, 0)

        wait_set(u0, v0, su0, sv0)
        prefetch(NCHUNK, TAIL, u0.at[pl.ds(0, TAIL)], v0.at[pl.ds(0, TAIL)],
                 su0, sv0)
        pltpu.make_async_copy(u_hbm.at[src_all.at[pl.ds(0, TAIL)]],
                              u0.at[pl.ds(0, TAIL)], su0).wait()
        pltpu.make_async_copy(v_hbm.at[dst_all.at[pl.ds(0, TAIL)]],
                              v0.at[pl.ds(0, TAIL)], sv0).wait()
        compute(TAIL, u0, v0, 0, unroll=1)
        store_scores(NCHUNK, TAIL)

    return sc_kernel


def kernel(h, edge_index, W1, b1, W2, b2):
    AT = W1[:, :H].T
    BT = W1[:, H:].T
    b1row = b1.reshape(1, H).astype(jnp.float32)
    U, V = _tc_precompute(h, AT, BT, b1row)

    ei = edge_index.astype(jnp.int32)
    src = ei[0]
    dst = ei[1]
    w2 = W2.reshape(H).astype(jnp.float32)
    b2vec = jnp.broadcast_to(b2.astype(jnp.float32), (16,))

    sc = _make_sc_kernel()
    return sc(U, V, src, dst, w2, b2vec)


# speedup vs baseline: 2.3207x; 2.3207x over previous
import functools
import jax
import jax.numpy as jnp
from jax import lax
from jax.experimental import pallas as pl
from jax.experimental.pallas import tpu as pltpu
from jax.experimental.pallas import tpu_sc as plsc

N_NODES = 10000
H = 128
E = 320000
NC = 2
NS = 16
NW = NC * NS
EPW = E // NW
C = 160
NCHUNK = EPW // C
TAIL = EPW - NCHUNK * C
NPAIR = NCHUNK // 2
G = C // 16


def _tc_precompute(h, AT, BT, b1row):
    BLK = 1000

    def body(h_ref, at_ref, bt_ref, b1_ref, u_ref, v_ref):
        hb = h_ref[...]
        u_ref[...] = jnp.dot(hb, at_ref[...], preferred_element_type=jnp.float32)
        v_ref[...] = (jnp.dot(hb, bt_ref[...], preferred_element_type=jnp.float32)
                      + b1_ref[...])

    return pl.pallas_call(
        body,
        grid=(N_NODES // BLK,),
        in_specs=[
            pl.BlockSpec((BLK, H), lambda i: (i, 0)),
            pl.BlockSpec((H, H), lambda i: (0, 0)),
            pl.BlockSpec((H, H), lambda i: (0, 0)),
            pl.BlockSpec((1, H), lambda i: (0, 0)),
        ],
        out_specs=[
            pl.BlockSpec((BLK, H), lambda i: (i, 0)),
            pl.BlockSpec((BLK, H), lambda i: (i, 0)),
        ],
        out_shape=[
            jax.ShapeDtypeStruct((N_NODES, H), jnp.float32),
            jax.ShapeDtypeStruct((N_NODES, H), jnp.float32),
        ],
    )(h, AT, BT, b1row)


def _make_sc_kernel():
    mesh = plsc.VectorSubcoreMesh(core_axis_name="c", subcore_axis_name="s")

    @functools.partial(
        pl.kernel,
        out_type=jax.ShapeDtypeStruct((E,), jnp.float32),
        mesh=mesh,
        compiler_params=pltpu.CompilerParams(needs_layout_passes=False),
        scratch_types=[
            pltpu.VMEM((EPW,), jnp.int32),
            pltpu.VMEM((EPW,), jnp.int32),
            pltpu.VMEM((C, H), jnp.float32),
            pltpu.VMEM((C, H), jnp.float32),
            pltpu.VMEM((C, H), jnp.float32),
            pltpu.VMEM((C, H), jnp.float32),
            pltpu.VMEM((2 * C,), jnp.float32),
            pltpu.VMEM((G * 256,), jnp.float32),
            pltpu.VMEM((H,), jnp.float32),
            pltpu.VMEM((16,), jnp.float32),
            pltpu.SemaphoreType.DMA,
            pltpu.SemaphoreType.DMA,
            pltpu.SemaphoreType.DMA,
            pltpu.SemaphoreType.DMA,
        ],
    )
    def sc_kernel(u_hbm, v_hbm, src_hbm, dst_hbm, w2_hbm, b2_hbm, out_hbm,
                  src_all, dst_all, u0, v0, u1, v1, scores_v, red_flat,
                  w2_v, b2_v, su0, sv0, su1, sv1):
        wid = lax.axis_index("s") * NC + lax.axis_index("c")
        base = wid * EPW

        pltpu.sync_copy(src_hbm.at[pl.ds(base, EPW)], src_all)
        pltpu.sync_copy(dst_hbm.at[pl.ds(base, EPW)], dst_all)
        pltpu.sync_copy(w2_hbm, w2_v)
        pltpu.sync_copy(b2_hbm, b2_v)
        w2v = [w2_v[pl.ds(16 * j, 16)] for j in range(8)]
        b2vec = b2_v[...]
        lane = lax.iota(jnp.int32, 16)

        def prefetch(c, n, ubuf, vbuf, su, sv):
            sl = src_all.at[pl.ds(c * C, n)]
            dl = dst_all.at[pl.ds(c * C, n)]
            pltpu.async_copy(u_hbm.at[sl], ubuf, su)
            pltpu.async_copy(v_hbm.at[dl], vbuf, sv)

        def wait_set(ubuf, vbuf, su, sv):
            pltpu.make_async_copy(u_hbm.at[src_all.at[pl.ds(0, C)]], ubuf, su).wait()
            pltpu.make_async_copy(v_hbm.at[dst_all.at[pl.ds(0, C)]], vbuf, sv).wait()

        def compute(n, ubuf, vbuf, soff, unroll=2):
            @plsc.parallel_loop(0, n // 16, unroll=unroll)
            def group_body(g):
                gbase = g * 16
                roff = g * 256
                for e in range(16):
                    row = gbase + e
                    acc = None
                    for j in range(8):
                        u = ubuf[row, pl.ds(16 * j, 16)]
                        v = vbuf[row, pl.ds(16 * j, 16)]
                        t = jnp.maximum(u + v, 0.0) * w2v[j]
                        acc = t if acc is None else acc + t
                    red_flat[pl.ds(roff + e * 16, 16)] = acc
                idx = lane * 16 + roff
                s0 = plsc.load_gather(red_flat, [idx])
                s1 = None
                for j in range(1, 16):
                    idx = idx + 1
                    t = plsc.load_gather(red_flat, [idx])
                    if j % 2:
                        s1 = t if s1 is None else s1 + t
                    else:
                        s0 = s0 + t
                scores_v[pl.ds(soff + gbase, 16)] = s0 + s1 + b2vec

        def store_scores(c, n):
            pltpu.sync_copy(scores_v.at[pl.ds(0, n)],
                            out_hbm.at[pl.ds(base + c * C, n)])

        prefetch(0, C, u0, v0, su0, sv0)

        def pair_body(i, carry):
            c0 = i * 2
            prefetch(c0 + 1, C, u1, v1, su1, sv1)
            wait_set(u0, v0, su0, sv0)
            compute(C, u0, v0, 0)
            cn = lax.rem(c0 + 2, NCHUNK)
            prefetch(cn, C, u0, v0, su0, sv0)
            wait_set(u1, v1, su1, sv1)
            compute(C, u1, v1, C)
            store_scores(c0, 2 * C)
            return carry

        lax.fori_loop(0, NPAIR, pair_body, 0)

        wait_set(u0, v0, su0, sv0)
        prefetch(NCHUNK, TAIL, u0.at[pl.ds(0, TAIL)], v0.at[pl.ds(0, TAIL)],
                 su0, sv0)
        pltpu.make_async_copy(u_hbm.at[src_all.at[pl.ds(0, TAIL)]],
                              u0.at[pl.ds(0, TAIL)], su0).wait()
        pltpu.make_async_copy(v_hbm.at[dst_all.at[pl.ds(0, TAIL)]],
                              v0.at[pl.ds(0, TAIL)], sv0).wait()
        compute(TAIL, u0, v0, 0, unroll=1)
        store_scores(NCHUNK, TAIL)

    return sc_kernel


def kernel(h, edge_index, W1, b1, W2, b2):
    AT = W1[:, :H].T
    BT = W1[:, H:].T
    b1row = b1.reshape(1, H).astype(jnp.float32)
    U, V = _tc_precompute(h, AT, BT, b1row)

    ei = edge_index.astype(jnp.int32)
    src = ei[0]
    dst = ei[1]
    w2 = W2.reshape(H).astype(jnp.float32)
    b2vec = jnp.broadcast_to(b2.astype(jnp.float32), (16,))

    sc = _make_sc_kernel()
    return sc(U, V, src, dst, w2, b2vec)
